# baseline (speedup 1.0000x reference)
import functools
import math

import jax
import jax.numpy as jnp
from jax import lax
from jax.experimental import pallas as pl
from jax.experimental.pallas import tpu as pltpu
from jax.experimental.pallas import tpu_sc as plsc

B, C, L, D = 4096, 20, 50, 64
CP = 24
LP = 56
MP = 64
CO = 32
NC, NS = 2, 16
NW = NC * NS
BW = B // NW
NV = D // 16


_GDN = lax.GatherDimensionNumbers(
    offset_dims=(), collapsed_slice_dims=(0,), start_index_map=(0,))


def _permute(v, idx):
    return lax.gather(v, idx[:, None], dimension_numbers=_GDN,
                      slice_sizes=(1,),
                      mode=lax.GatherScatterMode.PROMISE_IN_BOUNDS)


def _lanesum(v, perms):
    for p in perms:
        v = v + _permute(v, p)
    return v


def _body(emb_hbm, cdd_hbm, his_hbm, mask_hbm, out_hbm,
          cdd_idx_v, his_idx_v, mask_v, logits_v, his_rows, cdd_rows,
          sem_h, sem_c):
    wid = lax.axis_index("s") * NC + lax.axis_index("c")
    base = wid * BW

    pltpu.sync_copy(cdd_hbm.at[pl.ds(base, BW)], cdd_idx_v)
    pltpu.sync_copy(his_hbm.at[pl.ds(base, BW)], his_idx_v)
    pltpu.sync_copy(mask_hbm.at[pl.ds(base, BW)], mask_v)

    lane = lax.iota(jnp.int32, 16)
    perms = [lane ^ k for k in (1, 2, 4, 8)]

    def batch_body(bi, _):
        ch = pltpu.async_copy(emb_hbm.at[his_idx_v.at[bi]], his_rows, sem_h)
        cc = pltpu.async_copy(emb_hbm.at[cdd_idx_v.at[bi]], cdd_rows, sem_c)

        mvecs = [mask_v[bi, pl.ds(16 * g, 16)] for g in range(MP // 16)]
        msum_vec = mvecs[0]
        for g in range(1, MP // 16):
            msum_vec = msum_vec + mvecs[g]
        inv = 1.0 / (_lanesum(msum_vec, perms) + 1e-6)

        ch.wait()

        acc = [jnp.zeros((16,), jnp.float32) for _ in range(NV)]
        for l in range(L):
            m = mvecs[l // 16][l % 16]
            for j in range(NV):
                acc[j] = acc[j] + m * his_rows[l, pl.ds(16 * j, 16)]
        scale = inv * (1.0 / math.sqrt(D))
        user = [acc[j] * scale for j in range(NV)]

        cc.wait()

        rows = [jnp.zeros((16,), jnp.float32) for _ in range(CO // 16)]
        for c in range(C):
            dot = cdd_rows[c, pl.ds(0, 16)] * user[0]
            for j in range(1, NV):
                dot = dot + cdd_rows[c, pl.ds(16 * j, 16)] * user[j]
            s = _lanesum(dot, perms)
            rows[c // 16] = jnp.where(lane == (c % 16), s, rows[c // 16])
        for g in range(CO // 16):
            logits_v[bi, pl.ds(16 * g, 16)] = rows[g]
        return ()

    lax.fori_loop(0, BW, batch_body, ())

    pltpu.sync_copy(logits_v, out_hbm.at[pl.ds(base, BW)])


@functools.partial(
    pl.kernel,
    out_type=jax.ShapeDtypeStruct((B, CO), jnp.float32),
    mesh=plsc.VectorSubcoreMesh(core_axis_name="c", subcore_axis_name="s"),
    compiler_params=pltpu.CompilerParams(use_tc_tiling_on_sc=False),
    scratch_types=[
        pltpu.VMEM((BW, CP), jnp.int32),
        pltpu.VMEM((BW, LP), jnp.int32),
        pltpu.VMEM((BW, MP), jnp.float32),
        pltpu.VMEM((BW, CO), jnp.float32),
        pltpu.VMEM((LP, D), jnp.float32),
        pltpu.VMEM((CP, D), jnp.float32),
        pltpu.SemaphoreType.DMA,
        pltpu.SemaphoreType.DMA,
    ],
)
def _sc_two_tower(*args):
    _body(*args)


def kernel(news_embeddings, cdd_idx, his_idx, his_mask):
    cdd_p = jnp.pad(cdd_idx.astype(jnp.int32), ((0, 0), (0, CP - C)))
    his_p = jnp.pad(his_idx.astype(jnp.int32), ((0, 0), (0, LP - L)))
    mask_p = jnp.pad(his_mask, ((0, 0), (0, MP - L)))
    out = _sc_two_tower(news_embeddings, cdd_p, his_p, mask_p)
    return out[:, :C]
